# Initial kernel scaffold; baseline (speedup 1.0000x reference)
import jax
import jax.numpy as jnp
from jax.experimental import pallas as pl


def kernel(atom_pos, cell, pbc_offsets, atom_attr, edge_index, three_body_indices, num_three_body, num_triple_ij, num_atoms, num_bonds, num_graphs, W_embed, W_edge, Wsbf, Wk, Wg1, Wg2, We1, We2, Wer, Wa1, Wa2, War, F1, F2, F3, V1, V2, V3, scale, shift):
    raise NotImplementedError("write your pallas kernel here")


# speedup vs baseline: 1.0241x; 1.0241x over previous
import functools

import jax
import jax.numpy as jnp
from jax import lax
from jax.experimental import pallas as pl
from jax.experimental.pallas import tpu as pltpu

HID = 64
MAXN = 4
MAXL = 4
CUT = 5.0
CUT3 = 4.0
NLAYERS = 4
N = 10000
E = 160000
T = 160000
NPAD = 10240
NB = 1024
EB = 4000
TB = 4000

_f32 = jnp.float32


def _sigmoid(x):
    return jax.nn.sigmoid(x)


def _swish(x):
    return x * jax.nn.sigmoid(x)


def _bessel(r, c):
    rc = jnp.clip(r, 1e-6, None)
    out = []
    s = jnp.sqrt(2.0 / c)
    for n in range(1, MAXN + 1):
        out.append(s * jnp.sin((n * jnp.pi / c) * rc) / rc)
    return jnp.stack(out, axis=-1)


def _cutoff(r, c):
    x = jnp.clip(r / c, 0.0, 1.0)
    return 1.0 - 6.0 * x ** 5 + 15.0 * x ** 4 - 10.0 * x ** 3


def _k1_body(z_ref, wemb_ref, sc_ref, sh_ref, atoms_ref, scz_ref, shz_ref):
    z = z_ref[:, 0]
    oh = (jax.lax.broadcasted_iota(jnp.int32, (NB, 128), 1) == z[:, None]).astype(_f32)
    atoms_ref[...] = jnp.dot(oh, wemb_ref[...], preferred_element_type=_f32)
    scz_ref[...] = jnp.dot(oh, sc_ref[...], preferred_element_type=_f32)
    shz_ref[...] = jnp.dot(oh, sh_ref[...], preferred_element_type=_f32)


def _k1(z_pad, wemb_pad, scale_pad, shift_pad):
    return pl.pallas_call(
        _k1_body,
        grid=(NPAD // NB,),
        in_specs=[
            pl.BlockSpec((NB, 1), lambda i: (i, 0)),
            pl.BlockSpec((128, HID), lambda i: (0, 0)),
            pl.BlockSpec((128, 1), lambda i: (0, 0)),
            pl.BlockSpec((128, 1), lambda i: (0, 0)),
        ],
        out_specs=[
            pl.BlockSpec((NB, HID), lambda i: (i, 0)),
            pl.BlockSpec((NB, 1), lambda i: (i, 0)),
            pl.BlockSpec((NB, 1), lambda i: (i, 0)),
        ],
        out_shape=[
            jax.ShapeDtypeStruct((NPAD, HID), _f32),
            jax.ShapeDtypeStruct((NPAD, 1), _f32),
            jax.ShapeDtypeStruct((NPAD, 1), _f32),
        ],
    )(z_pad, wemb_pad, scale_pad, shift_pad)


def _k2_body(g0_ref, g1_ref, pbc_ref, cell_ref, wedge_ref, geom_ref, eattr_ref):
    off = jnp.dot(pbc_ref[...], cell_ref[...], preferred_element_type=_f32)
    d = g0_ref[...] - g1_ref[...] - off
    elen = jnp.sqrt(jnp.sum(d * d, axis=1))
    ez = _bessel(elen, CUT)
    c3 = _cutoff(elen, CUT3)
    geom_ref[...] = jnp.concatenate(
        [d[:, :3], elen[:, None], ez, c3[:, None], jnp.zeros((EB, 7), _f32)], axis=1)
    eattr_ref[...] = jnp.dot(ez, wedge_ref[...], preferred_element_type=_f32)


def _k2(g0, g1, pbc_pad, cell_pad, wedge):
    return pl.pallas_call(
        _k2_body,
        grid=(E // EB,),
        in_specs=[
            pl.BlockSpec((EB, 16), lambda i: (i, 0)),
            pl.BlockSpec((EB, 16), lambda i: (i, 0)),
            pl.BlockSpec((EB, 16), lambda i: (i, 0)),
            pl.BlockSpec((16, 16), lambda i: (0, 0)),
            pl.BlockSpec((MAXN, HID), lambda i: (0, 0)),
        ],
        out_specs=[
            pl.BlockSpec((EB, 16), lambda i: (i, 0)),
            pl.BlockSpec((EB, HID), lambda i: (i, 0)),
        ],
        out_shape=[
            jax.ShapeDtypeStruct((E, 16), _f32),
            jax.ShapeDtypeStruct((E, HID), _f32),
        ],
    )(g0, g1, pbc_pad, cell_pad, wedge)


def _k3_body(gj_ref, gk_ref, out_ref):
    gj = gj_ref[...]
    gk = gk_ref[...]
    rij = gj[:, 3]
    rik = gk[:, 3]
    dot = jnp.sum(gj[:, :3] * gk[:, :3], axis=1)
    cos = dot / jnp.clip(rij * rik, 1e-8, None)
    cos = jnp.clip(cos, -1.0 + 1e-7, 1.0 - 1e-7)
    leg = jnp.stack(
        [jnp.ones_like(cos), cos, 0.5 * (3.0 * cos ** 2 - 1.0),
         0.5 * (5.0 * cos ** 3 - 3.0 * cos)], axis=-1)
    rad = gk[:, 4:8]
    three = (rad[:, None, :] * leg[:, :, None]).reshape(TB, MAXL * MAXN)
    w3 = gj[:, 8] * gk[:, 8]
    out_ref[...] = three * w3[:, None]


def _k3(gj, gk):
    return pl.pallas_call(
        _k3_body,
        grid=(T // TB,),
        in_specs=[
            pl.BlockSpec((TB, 16), lambda i: (i, 0)),
            pl.BlockSpec((TB, 16), lambda i: (i, 0)),
        ],
        out_specs=pl.BlockSpec((TB, 16), lambda i: (i, 0)),
        out_shape=jax.ShapeDtypeStruct((T, 16), _f32),
    )(gj, gk)


def _k4_body(tw_ref, ak_ref, wsbf_ref, wk_ref, m3_ref):
    a = jnp.dot(tw_ref[...], wsbf_ref[...], preferred_element_type=_f32)
    b = _sigmoid(jnp.dot(ak_ref[...], wk_ref[...], preferred_element_type=_f32))
    m3_ref[...] = a * b


def _k4(three_w, ak, wsbf_l, wk_l):
    return pl.pallas_call(
        _k4_body,
        grid=(T // TB,),
        in_specs=[
            pl.BlockSpec((TB, 16), lambda i: (i, 0)),
            pl.BlockSpec((TB, HID), lambda i: (i, 0)),
            pl.BlockSpec((16, HID), lambda i: (0, 0)),
            pl.BlockSpec((HID, HID), lambda i: (0, 0)),
        ],
        out_specs=pl.BlockSpec((TB, HID), lambda i: (i, 0)),
        out_shape=jax.ShapeDtypeStruct((T, HID), _f32),
    )(three_w, ak, wsbf_l, wk_l)


def _k6_body(a0_ref, a1_ref, eattr_ref, agg_ref, geom_ref,
             wg1_ref, wg2_ref, we_ref, wa_ref, wer_ref, war_ref,
             eout_ref, msg_ref):
    agg = agg_ref[...]
    eattr = eattr_ref[...]
    a0 = a0_ref[...]
    a1 = a1_ref[...]
    geom = geom_ref[...]

    g1 = jnp.dot(agg, wg1_ref[...], preferred_element_type=_f32)
    g2 = jnp.dot(agg, wg2_ref[...], preferred_element_type=_f32)
    eattr = eattr + _swish(g1) * _sigmoid(g2)

    er = jnp.dot(geom, wer_ref[...], preferred_element_type=_f32)
    ar = jnp.dot(geom, war_ref[...], preferred_element_type=_f32)

    a01 = jnp.concatenate([a0, a1], axis=1)
    we = we_ref[...]
    wa = wa_ref[...]
    p = jnp.dot(a01, we[:128], preferred_element_type=_f32)
    e1 = p[:, :HID] + jnp.dot(eattr, we[128:192, :HID], preferred_element_type=_f32)
    e2 = p[:, HID:] + jnp.dot(eattr, we[128:192, HID:], preferred_element_type=_f32)
    eattr = eattr + _swish(e1) * _sigmoid(e2) * er
    eout_ref[...] = eattr

    q = jnp.dot(a01, wa[:128], preferred_element_type=_f32)
    m1 = q[:, :HID] + jnp.dot(eattr, wa[128:192, :HID], preferred_element_type=_f32)
    m2 = q[:, HID:] + jnp.dot(eattr, wa[128:192, HID:], preferred_element_type=_f32)
    msg_ref[...] = _swish(m1) * _sigmoid(m2) * ar


def _k6(a0, a1, eattr, agg, geom, wg1_l, wg2_l, we_l, wa_l, wer_l, war_l):
    return pl.pallas_call(
        _k6_body,
        grid=(E // EB,),
        in_specs=[
            pl.BlockSpec((EB, HID), lambda i: (i, 0)),
            pl.BlockSpec((EB, HID), lambda i: (i, 0)),
            pl.BlockSpec((EB, HID), lambda i: (i, 0)),
            pl.BlockSpec((EB, HID), lambda i: (i, 0)),
            pl.BlockSpec((EB, 16), lambda i: (i, 0)),
            pl.BlockSpec((HID, HID), lambda i: (0, 0)),
            pl.BlockSpec((HID, HID), lambda i: (0, 0)),
            pl.BlockSpec((192, 128), lambda i: (0, 0)),
            pl.BlockSpec((192, 128), lambda i: (0, 0)),
            pl.BlockSpec((16, HID), lambda i: (0, 0)),
            pl.BlockSpec((16, HID), lambda i: (0, 0)),
        ],
        out_specs=[
            pl.BlockSpec((EB, HID), lambda i: (i, 0)),
            pl.BlockSpec((EB, HID), lambda i: (i, 0)),
        ],
        out_shape=[
            jax.ShapeDtypeStruct((E, HID), _f32),
            jax.ShapeDtypeStruct((E, HID), _f32),
        ],
    )(a0, a1, eattr, agg, geom, wg1_l, wg2_l, we_l, wa_l, wer_l, war_l)


def _k9_body(atoms_ref, scz_ref, shz_ref, f1_ref, f2_ref, f3_ref,
             v1_ref, v2_ref, v3_ref, out_ref):
    a = atoms_ref[...]
    h = _swish(jnp.dot(a, f1_ref[...], preferred_element_type=_f32))
    h = _swish(jnp.dot(h, f2_ref[...], preferred_element_type=_f32))
    h = jnp.dot(h, f3_ref[...], preferred_element_type=_f32)
    g = _swish(jnp.dot(a, v1_ref[...], preferred_element_type=_f32))
    g = _swish(jnp.dot(g, v2_ref[...], preferred_element_type=_f32))
    g = _sigmoid(jnp.dot(g, v3_ref[...], preferred_element_type=_f32))
    e = h[:, 0] * g[:, 0] * scz_ref[:, 0] + shz_ref[:, 0]

    @pl.when(pl.program_id(0) == 0)
    def _():
        out_ref[...] = jnp.zeros_like(out_ref)

    out_ref[...] += jnp.sum(e)[None, None]


def _k9(atoms, scz, shz, F1, F2, F3, V1, V2, V3):
    return pl.pallas_call(
        _k9_body,
        grid=(NPAD // NB,),
        in_specs=[
            pl.BlockSpec((NB, HID), lambda i: (i, 0)),
            pl.BlockSpec((NB, 1), lambda i: (i, 0)),
            pl.BlockSpec((NB, 1), lambda i: (i, 0)),
            pl.BlockSpec((HID, HID), lambda i: (0, 0)),
            pl.BlockSpec((HID, HID), lambda i: (0, 0)),
            pl.BlockSpec((HID, 1), lambda i: (0, 0)),
            pl.BlockSpec((HID, HID), lambda i: (0, 0)),
            pl.BlockSpec((HID, HID), lambda i: (0, 0)),
            pl.BlockSpec((HID, 1), lambda i: (0, 0)),
        ],
        out_specs=pl.BlockSpec((1, 1), lambda i: (0, 0)),
        out_shape=jax.ShapeDtypeStruct((1, 1), _f32),
    )(atoms, scz, shz, F1, F2, F3, V1, V2, V3)


def _gather_rows(table, idx):
    return jnp.take(table, idx, axis=0)


def _segsum(rows, idx, nseg):
    return jax.ops.segment_sum(rows, idx, num_segments=nseg)


def kernel(atom_pos, cell, pbc_offsets, atom_attr, edge_index,
           three_body_indices, num_three_body, num_triple_ij, num_atoms,
           num_bonds, num_graphs, W_embed, W_edge, Wsbf, Wk, Wg1, Wg2,
           We1, We2, Wer, Wa1, Wa2, War, F1, F2, F3, V1, V2, V3,
           scale, shift):
    i0 = edge_index[0]
    i1 = edge_index[1]
    t0 = three_body_indices[:, 0]
    t1 = three_body_indices[:, 1]
    k_idx = jnp.take(i1, t1)

    z_pad = jnp.full((NPAD, 1), 127, jnp.int32).at[:N, 0].set(atom_attr[:, 0].astype(jnp.int32))
    wemb_pad = jnp.zeros((128, HID), _f32).at[:95].set(W_embed)
    scale_pad = jnp.zeros((128, 1), _f32).at[:95, 0].set(scale)
    shift_pad = jnp.zeros((128, 1), _f32).at[:95, 0].set(shift)
    pos_pad = jnp.zeros((N, 16), _f32).at[:, :3].set(atom_pos)
    pbc_pad = jnp.zeros((E, 16), _f32).at[:, :3].set(pbc_offsets)
    cell_pad = jnp.zeros((16, 16), _f32).at[:3, :3].set(cell[0])
    we_pack = jnp.concatenate([We1, We2], axis=2)
    wa_pack = jnp.concatenate([Wa1, Wa2], axis=2)
    wer_pad = jnp.zeros((NLAYERS, 16, HID), _f32).at[:, 4:8].set(Wer)
    war_pad = jnp.zeros((NLAYERS, 16, HID), _f32).at[:, 4:8].set(War)

    atoms, scz, shz = _k1(z_pad, wemb_pad, scale_pad, shift_pad)
    atoms = atoms[:N]

    g0 = _gather_rows(pos_pad, i0)
    g1 = _gather_rows(pos_pad, i1)
    geom, eattr = _k2(g0, g1, pbc_pad, cell_pad, W_edge)

    gj = _gather_rows(geom, t0)
    gk = _gather_rows(geom, t1)
    three_w = _k3(gj, gk)

    for l in range(NLAYERS):
        ak = _gather_rows(atoms, k_idx)
        m3 = _k4(three_w, ak, Wsbf[l], Wk[l])
        agg = _segsum(m3, t0, E)
        a0 = _gather_rows(atoms, i0)
        a1 = _gather_rows(atoms, i1)
        eattr, msg = _k6(a0, a1, eattr, agg, geom,
                         Wg1[l], Wg2[l], we_pack[l], wa_pack[l],
                         wer_pad[l], war_pad[l])
        atoms = atoms + _segsum(msg, i0, N)

    atoms_pad = jnp.zeros((NPAD, HID), _f32).at[:N].set(atoms)
    out = _k9(atoms_pad, scz, shz, F1, F2, F3, V1, V2, V3)
    return out[0]
